# baseline (speedup 1.0000x reference)
import functools

import jax
import jax.numpy as jnp
from jax import lax
from jax.experimental import pallas as pl
from jax.experimental.pallas import tpu as pltpu
from jax.experimental.pallas import tpu_sc as plsc

_LANES = 16
_NUM_CORES = 2
_NUM_SUBCORES = 16
_NUM_WORKERS = _NUM_CORES * _NUM_SUBCORES


def _fold_body(et_ref, wt_ref, b_ref, o_ref):
    at = lax.dot_general(
        wt_ref[...], et_ref[...],
        dimension_numbers=(((1,), (0,)), ((), ())),
        preferred_element_type=jnp.float32,
    ) + b_ref[...]
    bits = lax.bitcast_convert_type(at.astype(jnp.bfloat16), jnp.uint16)
    u = bits.astype(jnp.uint32)
    packed = (u[1:2, :] << 16) | u[0:1, :]
    o_ref[...] = lax.bitcast_convert_type(packed, jnp.int32)


def _fold_table(emb_t, w_t, b2d):
    D, V = emb_t.shape
    C = w_t.shape[0]
    RB = 30720
    return pl.pallas_call(
        _fold_body,
        grid=(pl.cdiv(V, RB),),
        in_specs=[
            pl.BlockSpec((D, RB), lambda i: (0, i)),
            pl.BlockSpec((C, D), lambda i: (0, 0)),
            pl.BlockSpec((C, 1), lambda i: (0, 0)),
        ],
        out_specs=pl.BlockSpec((1, RB), lambda i: (0, i)),
        out_shape=jax.ShapeDtypeStruct((1, V), jnp.int32),
    )(emb_t, w_t, b2d)


def _make_bag_kernel(V, C, B, L):
    assert C == 2 and B % (_NUM_WORKERS * _LANES) == 0 and L % 8 == 0
    bags_per_worker = B // _NUM_WORKERS
    n_chunks = 4
    chunk_bags = bags_per_worker // n_chunks
    groups_per_chunk = chunk_bags // _LANES
    inv_l = 1.0 / float(L)

    mesh = plsc.VectorSubcoreMesh(core_axis_name="c", subcore_axis_name="s")

    @functools.partial(
        pl.kernel,
        out_type=jax.ShapeDtypeStruct((C, B), jnp.float32),
        mesh=mesh,
        scratch_types=[
            pltpu.VMEM((V,), jnp.int32),
            pltpu.VMEM((L, chunk_bags), jnp.int32),
            pltpu.VMEM((L, chunk_bags), jnp.int32),
            pltpu.VMEM((C * bags_per_worker,), jnp.float32),
            pltpu.VMEM_SHARED((V,), jnp.int32),
            pltpu.SemaphoreType.DMA,
            pltpu.SemaphoreType.DMA,
            pltpu.SemaphoreType.DMA,
        ],
        compiler_params=pltpu.CompilerParams(needs_layout_passes=False),
    )
    def bag_kernel(a_hbm, t_hbm, o_hbm, a_v, t_v0, t_v1, o_v, a_sh,
                   sem0, sem1, sem_a):
        sid = lax.axis_index("s")
        wid = sid * _NUM_CORES + lax.axis_index("c")
        base = wid * bags_per_worker

        t_bufs = (t_v0, t_v1)
        sems = (sem0, sem1)

        def start_chunk(ci):
            src = t_hbm.at[:, pl.ds(base + ci * chunk_bags, chunk_bags)]
            return pltpu.async_copy(src, t_bufs[ci % 2], sems[ci % 2])

        pending = start_chunk(0)

        @pl.when(sid == 0)
        def _stage_table():
            pltpu.sync_copy(a_hbm.at[0, :], a_sh)

        plsc.subcore_barrier()
        a_cp = pltpu.async_copy(a_sh, a_v, sem_a)

        himask = jnp.full((_LANES,), -65536, jnp.int32)
        zacc = jnp.zeros((_LANES,), jnp.float32)

        a_cp.wait()

        for ci in range(n_chunks):
            pending.wait()
            if ci + 1 < n_chunks:
                pending = start_chunk(ci + 1)
            tbuf = t_bufs[ci % 2]

            def group(g, carry, tbuf=tbuf, ci=ci):
                col0 = g * _LANES

                def step(t, accs):
                    acc0, acc1 = accs
                    idx = tbuf[t, pl.ds(col0, _LANES)]
                    w = plsc.load_gather(a_v, [idx])
                    lo = plsc.bitcast(w << 16, jnp.float32)
                    hi = plsc.bitcast(w & himask, jnp.float32)
                    return acc0 + lo, acc1 + hi

                acc0, acc1 = lax.fori_loop(0, L, step, (zacc, zacc), unroll=8)
                obag = (ci * groups_per_chunk + g) * _LANES + lax.iota(jnp.int32, _LANES)
                plsc.store_scatter(o_v, [obag], acc0 * inv_l)
                plsc.store_scatter(o_v, [obag + bags_per_worker], acc1 * inv_l)
                return carry

            lax.fori_loop(0, groups_per_chunk, group, 0)

        pltpu.sync_copy(o_v.at[pl.ds(0, bags_per_worker)],
                        o_hbm.at[0, pl.ds(base, bags_per_worker)])
        pltpu.sync_copy(o_v.at[pl.ds(bags_per_worker, bags_per_worker)],
                        o_hbm.at[1, pl.ds(base, bags_per_worker)])

    return bag_kernel


def kernel(texts, emb_table, W, b):
    V, D = emb_table.shape
    C = W.shape[1]
    B, L = texts.shape
    A = _fold_table(emb_table.T, W.T, b.reshape(C, 1))
    bag = _make_bag_kernel(V, C, B, L)
    out_planar = bag(A, texts.astype(jnp.int32).T)
    return out_planar.T


# speedup vs baseline: 1.0097x; 1.0097x over previous
import functools

import jax
import jax.numpy as jnp
from jax import lax
from jax.experimental import pallas as pl
from jax.experimental.pallas import tpu as pltpu
from jax.experimental.pallas import tpu_sc as plsc

_LANES = 16
_NUM_CORES = 2
_NUM_SUBCORES = 16
_NUM_WORKERS = _NUM_CORES * _NUM_SUBCORES


def _fold_body(et_ref, wt_ref, b_ref, o_ref):
    at = lax.dot_general(
        wt_ref[...], et_ref[...],
        dimension_numbers=(((1,), (0,)), ((), ())),
        preferred_element_type=jnp.float32,
    ) + b_ref[...]
    bits = lax.bitcast_convert_type(at.astype(jnp.bfloat16), jnp.uint16)
    u = bits.astype(jnp.uint32)
    packed = (u[1:2, :] << 16) | u[0:1, :]
    o_ref[...] = lax.bitcast_convert_type(packed, jnp.int32)


def _fold_table(emb_t, w_t, b2d):
    D, V = emb_t.shape
    C = w_t.shape[0]
    RB = 15360
    return pl.pallas_call(
        _fold_body,
        grid=(pl.cdiv(V, RB),),
        in_specs=[
            pl.BlockSpec((D, RB), lambda i: (0, i)),
            pl.BlockSpec((C, D), lambda i: (0, 0)),
            pl.BlockSpec((C, 1), lambda i: (0, 0)),
        ],
        out_specs=pl.BlockSpec((1, RB), lambda i: (0, i)),
        out_shape=jax.ShapeDtypeStruct((1, V), jnp.int32),
    )(emb_t, w_t, b2d)


def _make_bag_kernel(V, C, B, L):
    assert C == 2 and B % (_NUM_WORKERS * _LANES) == 0 and L % 8 == 0
    bags_per_worker = B // _NUM_WORKERS
    n_chunks = 4
    chunk_bags = bags_per_worker // n_chunks
    groups_per_chunk = chunk_bags // _LANES
    inv_l = 1.0 / float(L)

    mesh = plsc.VectorSubcoreMesh(core_axis_name="c", subcore_axis_name="s")

    @functools.partial(
        pl.kernel,
        out_type=jax.ShapeDtypeStruct((C, B), jnp.float32),
        mesh=mesh,
        scratch_types=[
            pltpu.VMEM((V,), jnp.int32),
            pltpu.VMEM((L, chunk_bags), jnp.int32),
            pltpu.VMEM((L, chunk_bags), jnp.int32),
            pltpu.VMEM((C * bags_per_worker,), jnp.float32),
            pltpu.VMEM_SHARED((V,), jnp.int32),
            pltpu.SemaphoreType.DMA,
            pltpu.SemaphoreType.DMA,
            pltpu.SemaphoreType.DMA,
        ],
        compiler_params=pltpu.CompilerParams(needs_layout_passes=False),
    )
    def bag_kernel(a_hbm, t_hbm, o_hbm, a_v, t_v0, t_v1, o_v, a_sh,
                   sem0, sem1, sem_a):
        sid = lax.axis_index("s")
        wid = sid * _NUM_CORES + lax.axis_index("c")
        base = wid * bags_per_worker

        t_bufs = (t_v0, t_v1)
        sems = (sem0, sem1)

        def start_chunk(ci):
            src = t_hbm.at[:, pl.ds(base + ci * chunk_bags, chunk_bags)]
            return pltpu.async_copy(src, t_bufs[ci % 2], sems[ci % 2])

        pending = start_chunk(0)

        @pl.when(sid == 0)
        def _stage_table():
            pltpu.sync_copy(a_hbm.at[0, :], a_sh)

        plsc.subcore_barrier()
        a_cp = pltpu.async_copy(a_sh, a_v, sem_a)

        himask = jnp.full((_LANES,), -65536, jnp.int32)
        zacc = jnp.zeros((_LANES,), jnp.float32)

        a_cp.wait()

        for ci in range(n_chunks):
            pending.wait()
            if ci + 1 < n_chunks:
                pending = start_chunk(ci + 1)
            tbuf = t_bufs[ci % 2]

            def group(g, carry, tbuf=tbuf, ci=ci):
                col0 = g * _LANES

                def step(t, accs):
                    acc0, acc1 = accs
                    idx = tbuf[t, pl.ds(col0, _LANES)]
                    w = plsc.load_gather(a_v, [idx])
                    lo = plsc.bitcast(w << 16, jnp.float32)
                    hi = plsc.bitcast(w & himask, jnp.float32)
                    return acc0 + lo, acc1 + hi

                acc0, acc1 = lax.fori_loop(0, L, step, (zacc, zacc), unroll=8)
                obag = (ci * groups_per_chunk + g) * _LANES + lax.iota(jnp.int32, _LANES)
                plsc.store_scatter(o_v, [obag], acc0 * inv_l)
                plsc.store_scatter(o_v, [obag + bags_per_worker], acc1 * inv_l)
                return carry

            lax.fori_loop(0, groups_per_chunk, group, 0)

        pltpu.sync_copy(o_v.at[pl.ds(0, bags_per_worker)],
                        o_hbm.at[0, pl.ds(base, bags_per_worker)])
        pltpu.sync_copy(o_v.at[pl.ds(bags_per_worker, bags_per_worker)],
                        o_hbm.at[1, pl.ds(base, bags_per_worker)])

    return bag_kernel


def kernel(texts, emb_table, W, b):
    V, D = emb_table.shape
    C = W.shape[1]
    B, L = texts.shape
    A = _fold_table(emb_table.T, W.T, b.reshape(C, 1))
    bag = _make_bag_kernel(V, C, B, L)
    out_planar = bag(A, texts.astype(jnp.int32).T)
    return out_planar.T
